# baseline (speedup 1.0000x reference)
import jax
import jax.numpy as jnp
from jax import lax
from jax.experimental import pallas as pl
from jax.experimental.pallas import tpu as pltpu
from jax.experimental.pallas import tpu_sc as plsc

B, H, S, D, U = 8, 16, 2048, 128, 32
SLAB = S * D
CHUNK = U * D
HALF = B * H * SLAB


def _body(kc_hbm, vc_hbm, kn_hbm, vn_hbm, pos_hbm, out_hbm, pos_v, sem):
    c = lax.axis_index("c")
    s = lax.axis_index("s")
    wid = s * 2 + c
    b = wid // 4
    q = wid % 4

    pltpu.sync_copy(pos_hbm, pos_v)
    pos_b = pos_v[wid][0]

    copies = []
    for j in range(4):
        h = q * 4 + j
        off = (b * H + h) * SLAB
        copies.append(pltpu.async_copy(
            kc_hbm.at[pl.ds(off, SLAB)], out_hbm.at[pl.ds(off, SLAB)], sem))
        copies.append(pltpu.async_copy(
            vc_hbm.at[pl.ds(off, SLAB)], out_hbm.at[pl.ds(HALF + off, SLAB)],
            sem))
    for cp in copies:
        cp.wait()

    patches = []
    for j in range(4):
        h = q * 4 + j
        noff = (b * H + h) * CHUNK
        doff = (b * H + h) * SLAB + pos_b * D
        patches.append(pltpu.async_copy(
            kn_hbm.at[pl.ds(noff, CHUNK)], out_hbm.at[pl.ds(doff, CHUNK)],
            sem))
        patches.append(pltpu.async_copy(
            vn_hbm.at[pl.ds(noff, CHUNK)],
            out_hbm.at[pl.ds(HALF + doff, CHUNK)], sem))
    for p in patches:
        p.wait()


def kernel(k_new, v_new, cache_seqlens, qcache_seqlens, k_cache_buf, v_cache_buf):
    pos = (cache_seqlens - qcache_seqlens).astype(jnp.int32)
    wid_to_b = jnp.arange(32, dtype=jnp.int32) // 4
    pos_by_worker = jnp.broadcast_to(pos[wid_to_b][:, None], (32, 16))
    mesh = plsc.VectorSubcoreMesh(core_axis_name="c", subcore_axis_name="s")
    out_flat = pl.kernel(
        _body,
        out_type=jax.ShapeDtypeStruct((2 * HALF,), jnp.float32),
        mesh=mesh,
        scratch_types=[
            pltpu.VMEM((32, 16), jnp.int32),
            pltpu.SemaphoreType.DMA,
        ],
    )(
        k_cache_buf.reshape(-1),
        v_cache_buf.reshape(-1),
        k_new.reshape(-1),
        v_new.reshape(-1),
        pos_by_worker,
    )
    return out_flat.reshape(2, B, H, S, D)


# speedup vs baseline: 36.7338x; 36.7338x over previous
import jax
import jax.numpy as jnp
from jax import lax
from jax.experimental import pallas as pl
from jax.experimental.pallas import tpu as pltpu
from jax.experimental.pallas import tpu_sc as plsc

B, H, S, D, U = 8, 16, 2048, 128, 32
SLAB = S * D
CHUNK = U * D
HALF = B * H * SLAB
CH = 32768
NCH = SLAB // CH


def _body(kc_hbm, vc_hbm, kn_hbm, vn_hbm, pos_hbm, out_hbm,
          pos_v, buf0, buf1, sg0, sg1, ss0, ss1):
    c = lax.axis_index("c")
    s = lax.axis_index("s")
    wid = s * 2 + c
    b = wid // 4
    q = wid % 4

    pltpu.sync_copy(pos_hbm, pos_v)
    pos_b = pos_v[wid][0]

    for j in range(4):
        h = q * 4 + j
        base = (b * H + h) * SLAB
        for kv, src in ((0, kc_hbm), (1, vc_hbm)):
            dbase = kv * HALF + base
            g0 = pltpu.async_copy(src.at[pl.ds(base, CH)], buf0, sg0)
            g1 = pltpu.async_copy(src.at[pl.ds(base + CH, CH)], buf1, sg1)
            g0.wait()
            pltpu.async_copy(buf0, out_hbm.at[pl.ds(dbase, CH)], ss0)
            g1.wait()
            pltpu.async_copy(buf1, out_hbm.at[pl.ds(dbase + CH, CH)], ss1)

            def step(t, _):
                off = 2 * t * CH
                pltpu.make_async_copy(
                    buf0, out_hbm.at[pl.ds(dbase, CH)], ss0).wait()
                pltpu.async_copy(
                    src.at[pl.ds(base + off, CH)], buf0, sg0).wait()
                pltpu.async_copy(
                    buf0, out_hbm.at[pl.ds(dbase + off, CH)], ss0)
                pltpu.make_async_copy(
                    buf1, out_hbm.at[pl.ds(dbase, CH)], ss1).wait()
                pltpu.async_copy(
                    src.at[pl.ds(base + off + CH, CH)], buf1, sg1).wait()
                pltpu.async_copy(
                    buf1, out_hbm.at[pl.ds(dbase + off + CH, CH)], ss1)
                return _

            lax.fori_loop(1, NCH // 2, step, None)
            pltpu.make_async_copy(buf0, out_hbm.at[pl.ds(dbase, CH)],
                                  ss0).wait()
            pltpu.make_async_copy(buf1, out_hbm.at[pl.ds(dbase, CH)],
                                  ss1).wait()

    nbuf = buf0.at[pl.ds(0, CHUNK)]
    for j in range(4):
        h = q * 4 + j
        noff = (b * H + h) * CHUNK
        doff = (b * H + h) * SLAB + pos_b * D
        for kv, src in ((0, kn_hbm), (1, vn_hbm)):
            pltpu.async_copy(src.at[pl.ds(noff, CHUNK)], nbuf, sg0).wait()
            pltpu.async_copy(
                nbuf, out_hbm.at[pl.ds(kv * HALF + doff, CHUNK)], ss0).wait()


def kernel(k_new, v_new, cache_seqlens, qcache_seqlens, k_cache_buf, v_cache_buf):
    pos = (cache_seqlens - qcache_seqlens).astype(jnp.int32)
    wid_to_b = jnp.arange(32, dtype=jnp.int32) // 4
    pos_by_worker = jnp.broadcast_to(pos[wid_to_b][:, None], (32, 16))
    mesh = plsc.VectorSubcoreMesh(core_axis_name="c", subcore_axis_name="s")
    out_flat = pl.kernel(
        _body,
        out_type=jax.ShapeDtypeStruct((2 * HALF,), jnp.float32),
        mesh=mesh,
        scratch_types=[
            pltpu.VMEM((32, 16), jnp.int32),
            pltpu.VMEM((CH,), jnp.float32),
            pltpu.VMEM((CH,), jnp.float32),
            pltpu.SemaphoreType.DMA,
            pltpu.SemaphoreType.DMA,
            pltpu.SemaphoreType.DMA,
            pltpu.SemaphoreType.DMA,
        ],
    )(
        k_cache_buf.reshape(-1),
        v_cache_buf.reshape(-1),
        k_new.reshape(-1),
        v_new.reshape(-1),
        pos_by_worker,
    )
    return out_flat.reshape(2, B, H, S, D)


# speedup vs baseline: 38.6449x; 1.0520x over previous
import jax
import jax.numpy as jnp
from jax import lax
from jax.experimental import pallas as pl
from jax.experimental.pallas import tpu as pltpu
from jax.experimental.pallas import tpu_sc as plsc

B, H, S, D, U = 8, 16, 2048, 128, 32
SLAB = S * D
PCHUNK = U * D
HALF = B * H * SLAB
CH = 16384
CHPS = SLAB // CH
NCK = 4 * CHPS


def _body(kc_hbm, vc_hbm, kn_hbm, vn_hbm, pos_hbm, out_hbm,
          pos_v, b0, b1, b2, b3, sg0, sg1, sg2, sg3, ss0, ss1, ss2, ss3):
    c = lax.axis_index("c")
    s = lax.axis_index("s")
    wid = s * 2 + c
    b = wid // 4
    q = wid % 4

    pltpu.sync_copy(pos_hbm, pos_v)
    pos_b = pos_v[wid][0]

    bufs = (b0, b1, b2, b3)
    sg = (sg0, sg1, sg2, sg3)
    ss = (ss0, ss1, ss2, ss3)

    def phase(src, kvhalf):
        def soff(ch):
            return (b * H + q * 4 + ch // CHPS) * SLAB + (ch % CHPS) * CH

        def gather(ch, k):
            pltpu.async_copy(src.at[pl.ds(soff(ch), CH)], bufs[k], sg[k])

        def scatter(ch, k):
            pltpu.async_copy(
                bufs[k], out_hbm.at[pl.ds(kvhalf + soff(ch), CH)], ss[k])

        def wait_g(k):
            pltpu.make_async_copy(src.at[pl.ds(0, CH)], bufs[k], sg[k]).wait()

        def wait_s(k):
            pltpu.make_async_copy(
                bufs[k], out_hbm.at[pl.ds(0, CH)], ss[k]).wait()

        gather(0, 0)
        gather(1, 1)
        wait_g(0); scatter(0, 0); gather(2, 2)
        wait_g(1); scatter(1, 1); gather(3, 3)
        wait_g(2); scatter(2, 2); wait_s(0); gather(4, 0)
        wait_g(3); scatter(3, 3); wait_s(1); gather(5, 1)

        def body(i, carry):
            for k in range(4):
                ch = 4 * i + k
                kn = (k + 2) % 4
                wait_g(k); scatter(ch, k); wait_s(kn); gather(ch + 2, kn)
            return carry

        lax.fori_loop(1, (NCK - 4) // 4, body, None)

        wait_g(0); scatter(NCK - 4, 0); wait_s(2); gather(NCK - 2, 2)
        wait_g(1); scatter(NCK - 3, 1); wait_s(3); gather(NCK - 1, 3)
        wait_g(2); scatter(NCK - 2, 2)
        wait_g(3); scatter(NCK - 1, 3)
        for k in range(4):
            wait_s(k)

    phase(kc_hbm, 0)
    phase(vc_hbm, HALF)

    nbuf = b0.at[pl.ds(0, PCHUNK)]
    for j in range(4):
        h = q * 4 + j
        noff = (b * H + h) * PCHUNK
        doff = (b * H + h) * SLAB + pos_b * D
        for kvhalf, src in ((0, kn_hbm), (HALF, vn_hbm)):
            pltpu.async_copy(src.at[pl.ds(noff, PCHUNK)], nbuf, sg0).wait()
            pltpu.async_copy(
                nbuf, out_hbm.at[pl.ds(kvhalf + doff, PCHUNK)], ss0).wait()


def kernel(k_new, v_new, cache_seqlens, qcache_seqlens, k_cache_buf, v_cache_buf):
    pos = (cache_seqlens - qcache_seqlens).astype(jnp.int32)
    wid_to_b = jnp.arange(32, dtype=jnp.int32) // 4
    pos_by_worker = jnp.broadcast_to(pos[wid_to_b][:, None], (32, 16))
    mesh = plsc.VectorSubcoreMesh(core_axis_name="c", subcore_axis_name="s")
    out_flat = pl.kernel(
        _body,
        out_type=jax.ShapeDtypeStruct((2 * HALF,), jnp.float32),
        mesh=mesh,
        scratch_types=[
            pltpu.VMEM((32, 16), jnp.int32),
            pltpu.VMEM((CH,), jnp.float32),
            pltpu.VMEM((CH,), jnp.float32),
            pltpu.VMEM((CH,), jnp.float32),
            pltpu.VMEM((CH,), jnp.float32),
            pltpu.SemaphoreType.DMA,
            pltpu.SemaphoreType.DMA,
            pltpu.SemaphoreType.DMA,
            pltpu.SemaphoreType.DMA,
            pltpu.SemaphoreType.DMA,
            pltpu.SemaphoreType.DMA,
            pltpu.SemaphoreType.DMA,
            pltpu.SemaphoreType.DMA,
        ],
    )(
        k_cache_buf.reshape(-1),
        v_cache_buf.reshape(-1),
        k_new.reshape(-1),
        v_new.reshape(-1),
        pos_by_worker,
    )
    return out_flat.reshape(2, B, H, S, D)


# speedup vs baseline: 39.8228x; 1.0305x over previous
import jax
import jax.numpy as jnp
from jax import lax
from jax.experimental import pallas as pl
from jax.experimental.pallas import tpu as pltpu
from jax.experimental.pallas import tpu_sc as plsc

B, H, S, D, U = 8, 16, 2048, 128, 32
SLAB = S * D
PCHUNK = U * D
HALF = B * H * SLAB
CH = 16384
CHPS = SLAB // CH
NCK = 4 * CHPS


def _body(kc_hbm, vc_hbm, kn_hbm, vn_hbm, pos_hbm, out_hbm,
          pos_v, b0, b1, b2, b3, pb, sg0, sg1, sg2, sg3, ss0, ss1, ss2, ss3,
          sp):
    c = lax.axis_index("c")
    s = lax.axis_index("s")
    wid = s * 2 + c
    b = wid // 4
    q = wid % 4

    pltpu.sync_copy(pos_hbm, pos_v)
    pos_b = pos_v[wid][0]

    for j in range(4):
        noff = (b * H + q * 4 + j) * PCHUNK
        pltpu.async_copy(kn_hbm.at[pl.ds(noff, PCHUNK)],
                         pb.at[pl.ds((2 * j) * PCHUNK, PCHUNK)], sp)
        pltpu.async_copy(vn_hbm.at[pl.ds(noff, PCHUNK)],
                         pb.at[pl.ds((2 * j + 1) * PCHUNK, PCHUNK)], sp)

    bufs = (b0, b1, b2, b3)
    sg = (sg0, sg1, sg2, sg3)
    ss = (ss0, ss1, ss2, ss3)

    def phase(src, kvhalf):
        def soff(ch):
            return (b * H + q * 4 + ch // CHPS) * SLAB + (ch % CHPS) * CH

        def gather(ch, k):
            pltpu.async_copy(src.at[pl.ds(soff(ch), CH)], bufs[k], sg[k])

        def scatter(ch, k):
            pltpu.async_copy(
                bufs[k], out_hbm.at[pl.ds(kvhalf + soff(ch), CH)], ss[k])

        def wait_g(k):
            pltpu.make_async_copy(src.at[pl.ds(0, CH)], bufs[k], sg[k]).wait()

        def wait_s(k):
            pltpu.make_async_copy(
                bufs[k], out_hbm.at[pl.ds(0, CH)], ss[k]).wait()

        gather(0, 0)
        gather(1, 1)
        wait_g(0); scatter(0, 0); gather(2, 2)
        wait_g(1); scatter(1, 1); gather(3, 3)
        wait_g(2); scatter(2, 2); wait_s(0); gather(4, 0)
        wait_g(3); scatter(3, 3); wait_s(1); gather(5, 1)

        def body(i, carry):
            for k in range(4):
                ch = 4 * i + k
                kn = (k + 2) % 4
                wait_g(k); scatter(ch, k); wait_s(kn); gather(ch + 2, kn)
            return carry

        lax.fori_loop(1, (NCK - 4) // 4, body, None)

        wait_g(0); scatter(NCK - 4, 0); wait_s(2); gather(NCK - 2, 2)
        wait_g(1); scatter(NCK - 3, 1); wait_s(3); gather(NCK - 1, 3)
        wait_g(2); scatter(NCK - 2, 2)
        wait_g(3); scatter(NCK - 1, 3)
        for k in range(4):
            wait_s(k)

    phase(kc_hbm, 0)
    phase(vc_hbm, HALF)

    pltpu.make_async_copy(kn_hbm.at[pl.ds(0, 8 * PCHUNK)], pb, sp).wait()
    for j in range(4):
        doff = (b * H + q * 4 + j) * SLAB + pos_b * D
        pltpu.async_copy(pb.at[pl.ds((2 * j) * PCHUNK, PCHUNK)],
                         out_hbm.at[pl.ds(doff, PCHUNK)], sp)
        pltpu.async_copy(pb.at[pl.ds((2 * j + 1) * PCHUNK, PCHUNK)],
                         out_hbm.at[pl.ds(HALF + doff, PCHUNK)], sp)
    pltpu.make_async_copy(pb, kn_hbm.at[pl.ds(0, 8 * PCHUNK)], sp).wait()


def kernel(k_new, v_new, cache_seqlens, qcache_seqlens, k_cache_buf, v_cache_buf):
    pos = (cache_seqlens - qcache_seqlens).astype(jnp.int32)
    wid_to_b = jnp.arange(32, dtype=jnp.int32) // 4
    pos_by_worker = jnp.broadcast_to(pos[wid_to_b][:, None], (32, 16))
    mesh = plsc.VectorSubcoreMesh(core_axis_name="c", subcore_axis_name="s")
    out_flat = pl.kernel(
        _body,
        out_type=jax.ShapeDtypeStruct((2 * HALF,), jnp.float32),
        mesh=mesh,
        scratch_types=[
            pltpu.VMEM((32, 16), jnp.int32),
            pltpu.VMEM((CH,), jnp.float32),
            pltpu.VMEM((CH,), jnp.float32),
            pltpu.VMEM((CH,), jnp.float32),
            pltpu.VMEM((CH,), jnp.float32),
            pltpu.VMEM((8 * PCHUNK,), jnp.float32),
            pltpu.SemaphoreType.DMA,
            pltpu.SemaphoreType.DMA,
            pltpu.SemaphoreType.DMA,
            pltpu.SemaphoreType.DMA,
            pltpu.SemaphoreType.DMA,
            pltpu.SemaphoreType.DMA,
            pltpu.SemaphoreType.DMA,
            pltpu.SemaphoreType.DMA,
            pltpu.SemaphoreType.DMA,
        ],
    )(
        k_cache_buf.reshape(-1),
        v_cache_buf.reshape(-1),
        k_new.reshape(-1),
        v_new.reshape(-1),
        pos_by_worker,
    )
    return out_flat.reshape(2, B, H, S, D)
